# Initial kernel scaffold; baseline (speedup 1.0000x reference)
import jax
import jax.numpy as jnp
from jax.experimental import pallas as pl


def kernel(x, edge_index, W1, b1, W2, b2):
    raise NotImplementedError("write your pallas kernel here")


# speedup vs baseline: 27.7598x; 27.7598x over previous
import functools

import jax
import jax.numpy as jnp
from jax import lax
from jax.experimental import pallas as pl
from jax.experimental.pallas import tpu as pltpu
from jax.experimental.pallas import tpu_sc as plsc

N_NODES = 10000
N_PAD = 10240
D_IN = 128
D_H = 64
D_OUT = 16

NC = 2
NS = 16
NW = NC * NS
CHUNK = 128
RPT = N_PAD // NS

_f32 = jnp.float32
_mesh = plsc.VectorSubcoreMesh(core_axis_name="c", subcore_axis_name="s")
_sc_params = pltpu.CompilerParams(use_tc_tiling_on_sc=False)


def _deg_body(kch, dst_hbm, ones_hbm, zero_hbm, out_hbm,
              dst_v, ones_v, acc, sem):
    c = lax.axis_index("c")
    s = lax.axis_index("s")
    wid = c * NS + s
    rows = pl.ds(s * RPT, RPT)
    pltpu.sync_copy(zero_hbm.at[rows], acc.at[rows])
    pltpu.sync_copy(ones_hbm, ones_v)
    pltpu.sync_copy(dst_hbm.at[pl.ds(wid * kch, kch)], dst_v)
    plsc.subcore_barrier()

    def body(j, carry):
        pltpu.async_copy(ones_v, acc.at[dst_v.at[j]], sem, add=True).wait()
        return carry

    lax.fori_loop(0, kch, body, 0)
    plsc.subcore_barrier()
    pltpu.sync_copy(acc.at[rows], out_hbm.at[c].at[rows])


def _sc_degree(dst_rows, kch):
    kern = functools.partial(
        pl.kernel,
        out_type=jax.ShapeDtypeStruct((NC, N_PAD, 16), _f32),
        mesh=_mesh,
        scratch_types=[
            pltpu.VMEM((kch, CHUNK), jnp.int32),
            pltpu.VMEM((CHUNK, 16), _f32),
            pltpu.VMEM_SHARED((N_PAD, 16), _f32),
            pltpu.SemaphoreType.DMA,
        ],
        compiler_params=_sc_params,
    )(functools.partial(_deg_body, kch))
    ones = jnp.ones((CHUNK, 16), _f32)
    zeros = jnp.zeros((N_PAD, 16), _f32)
    return kern(dst_rows, ones, zeros)


def _agg_body(kch, d, g_hbm, src_hbm, dst_hbm, zero_hbm, out_hbm,
              src_v, dst_v, buf0, buf1, acc, sem0, sem1):
    c = lax.axis_index("c")
    s = lax.axis_index("s")
    wid = c * NS + s
    rows = pl.ds(s * RPT, RPT)
    pltpu.sync_copy(zero_hbm.at[rows], acc.at[rows])
    pltpu.sync_copy(src_hbm.at[pl.ds(wid * kch, kch)], src_v)
    pltpu.sync_copy(dst_hbm.at[pl.ds(wid * kch, kch)], dst_v)
    plsc.subcore_barrier()

    pltpu.async_copy(g_hbm.at[src_v.at[0]], buf0, sem0)

    def body(i, carry):
        j0 = 2 * i
        j1 = 2 * i + 1
        cp1 = pltpu.async_copy(g_hbm.at[src_v.at[j1]], buf1, sem1)
        pltpu.make_async_copy(g_hbm.at[src_v.at[j0]], buf0, sem0).wait()
        pltpu.sync_copy(buf0, acc.at[dst_v.at[j0]], add=True)

        @pl.when(j1 + 1 < kch)
        def _():
            pltpu.async_copy(g_hbm.at[src_v.at[j1 + 1]], buf0, sem0)

        cp1.wait()
        pltpu.sync_copy(buf1, acc.at[dst_v.at[j1]], add=True)
        return carry

    lax.fori_loop(0, kch // 2, body, 0)
    plsc.subcore_barrier()
    pltpu.sync_copy(acc.at[rows], out_hbm.at[c].at[rows])


def _sc_aggregate(g, src_rows, dst_rows, kch, d):
    kern = functools.partial(
        pl.kernel,
        out_type=jax.ShapeDtypeStruct((NC, N_PAD, d), _f32),
        mesh=_mesh,
        scratch_types=[
            pltpu.VMEM((kch, CHUNK), jnp.int32),
            pltpu.VMEM((kch, CHUNK), jnp.int32),
            pltpu.VMEM((CHUNK, d), _f32),
            pltpu.VMEM((CHUNK, d), _f32),
            pltpu.VMEM_SHARED((N_PAD, d), _f32),
            pltpu.SemaphoreType.DMA,
            pltpu.SemaphoreType.DMA,
        ],
        compiler_params=_sc_params,
    )(functools.partial(_agg_body, kch, d))
    zeros = jnp.zeros((N_PAD, d), _f32)
    return kern(g, src_rows, dst_rows, zeros)


def _dinv_of(degp_ref):
    p = degp_ref[...]
    deg = p[0, :, 0:1] + p[1, :, 0:1] + 1.0
    return lax.rsqrt(deg)


def _layer1_body(x_ref, w1_ref, degp_ref, g1_ref):
    dinv = _dinv_of(degp_ref)
    h = jnp.dot(x_ref[...], w1_ref[...], preferred_element_type=_f32)
    g1_ref[...] = h * dinv


def _layer2_body(aggp_ref, g1_ref, degp_ref, b1_ref, w2_ref, g2_ref):
    dinv = _dinv_of(degp_ref)
    agg = aggp_ref[0] + aggp_ref[1] + g1_ref[...]
    out1 = agg * dinv + b1_ref[...]
    r = jnp.maximum(out1, 0.0)
    h2 = jnp.dot(r, w2_ref[...], preferred_element_type=_f32)
    g2_ref[...] = h2 * dinv


def _final_body(aggp_ref, g2_ref, degp_ref, b2_ref, out_ref):
    dinv = _dinv_of(degp_ref)
    agg = aggp_ref[0] + aggp_ref[1] + g2_ref[...]
    out_ref[...] = agg * dinv + b2_ref[...]


_GRID = N_PAD // 128


def _tc_layer1(x_pad, W1, degp):
    return pl.pallas_call(
        _layer1_body,
        grid=(_GRID,),
        in_specs=[
            pl.BlockSpec((128, D_IN), lambda i: (i, 0)),
            pl.BlockSpec((D_IN, D_H), lambda i: (0, 0)),
            pl.BlockSpec((NC, 128, 16), lambda i: (0, i, 0)),
        ],
        out_specs=pl.BlockSpec((128, D_H), lambda i: (i, 0)),
        out_shape=jax.ShapeDtypeStruct((N_PAD, D_H), _f32),
    )(x_pad, W1, degp)


def _tc_layer2(aggp1, g1, degp, b1, W2):
    return pl.pallas_call(
        _layer2_body,
        grid=(_GRID,),
        in_specs=[
            pl.BlockSpec((NC, 128, D_H), lambda i: (0, i, 0)),
            pl.BlockSpec((128, D_H), lambda i: (i, 0)),
            pl.BlockSpec((NC, 128, 16), lambda i: (0, i, 0)),
            pl.BlockSpec((1, D_H), lambda i: (0, 0)),
            pl.BlockSpec((D_H, D_OUT), lambda i: (0, 0)),
        ],
        out_specs=pl.BlockSpec((128, D_OUT), lambda i: (i, 0)),
        out_shape=jax.ShapeDtypeStruct((N_PAD, D_OUT), _f32),
    )(aggp1, g1, degp, b1, W2)


def _tc_final(aggp2, g2, degp, b2):
    return pl.pallas_call(
        _final_body,
        grid=(_GRID,),
        in_specs=[
            pl.BlockSpec((NC, 128, D_OUT), lambda i: (0, i, 0)),
            pl.BlockSpec((128, D_OUT), lambda i: (i, 0)),
            pl.BlockSpec((NC, 128, 16), lambda i: (0, i, 0)),
            pl.BlockSpec((1, D_OUT), lambda i: (0, 0)),
        ],
        out_specs=pl.BlockSpec((128, D_OUT), lambda i: (i, 0)),
        out_shape=jax.ShapeDtypeStruct((N_PAD, D_OUT), _f32),
    )(aggp2, g2, degp, b2)


def kernel(x, edge_index, W1, b1, W2, b2):
    n, e = x.shape[0], edge_index.shape[1]
    kch8 = -(-e // (NW * CHUNK * 8)) * 8
    epw = kch8 * CHUNK
    e_pad = NW * epw
    kch = epw // CHUNK

    pad = N_NODES + (jnp.arange(e_pad - e, dtype=jnp.int32)
                     % (N_PAD - N_NODES))
    src_rows = jnp.concatenate([edge_index[0], pad]).reshape(-1, CHUNK)
    dst_rows = jnp.concatenate([edge_index[1], pad]).reshape(-1, CHUNK)

    x_pad = jnp.zeros((N_PAD, D_IN), _f32).at[:n].set(x)

    degp = _sc_degree(dst_rows, kch)
    g1 = _tc_layer1(x_pad, W1, degp)
    aggp1 = _sc_aggregate(g1, src_rows, dst_rows, kch, D_H)
    g2 = _tc_layer2(aggp1, g1, degp, b1.reshape(1, D_H), W2)
    aggp2 = _sc_aggregate(g2, src_rows, dst_rows, kch, D_OUT)
    out = _tc_final(aggp2, g2, degp, b2.reshape(1, D_OUT))
    return out[:n]


# speedup vs baseline: 41.0440x; 1.4785x over previous
import functools

import jax
import jax.numpy as jnp
import numpy as np
from jax import lax
from jax.experimental import pallas as pl
from jax.experimental.pallas import tpu as pltpu
from jax.experimental.pallas import tpu_sc as plsc

N_NODES = 10000
N_PAD = 10240
D_IN = 128
D_H = 64
D_OUT = 16

NC = 2
NS = 16
NW = NC * NS
CHUNK = 128
RPT = N_PAD // NS

_f32 = jnp.float32
_mesh = plsc.VectorSubcoreMesh(core_axis_name="c", subcore_axis_name="s")
_sc_params = pltpu.CompilerParams(use_tc_tiling_on_sc=False)


def _deg_body(kch, dst_hbm, ones_hbm, zero_hbm, out_hbm,
              dst_v, ones_v, acc, sem):
    c = lax.axis_index("c")
    s = lax.axis_index("s")
    wid = c * NS + s
    rows = pl.ds(s * RPT, RPT)
    pltpu.sync_copy(zero_hbm.at[rows], acc.at[rows])
    pltpu.sync_copy(ones_hbm, ones_v)
    pltpu.sync_copy(dst_hbm.at[pl.ds(wid * kch, kch)], dst_v)
    plsc.subcore_barrier()

    def body(j, carry):
        pltpu.async_copy(ones_v, acc.at[dst_v.at[j]], sem, add=True).wait()
        return carry

    lax.fori_loop(0, kch, body, 0)
    plsc.subcore_barrier()
    pltpu.sync_copy(acc.at[rows], out_hbm.at[c].at[rows])


def _sc_degree(dst_rows, kch):
    kern = functools.partial(
        pl.kernel,
        out_type=jax.ShapeDtypeStruct((NC, N_PAD, 16), _f32),
        mesh=_mesh,
        scratch_types=[
            pltpu.VMEM((kch, CHUNK), jnp.int32),
            pltpu.VMEM((CHUNK, 16), _f32),
            pltpu.VMEM_SHARED((N_PAD, 16), _f32),
            pltpu.SemaphoreType.DMA,
        ],
        compiler_params=_sc_params,
    )(functools.partial(_deg_body, kch))
    ones = jnp.ones((CHUNK, 16), _f32)
    zeros = jnp.zeros((N_PAD, 16), _f32)
    return kern(dst_rows, ones, zeros)


def _agg_body(kch, d, g_hbm, src_hbm, dst_hbm, zero_hbm, out_hbm,
              src_v, dst_v, buf0, buf1, acc, sem0, sem1):
    c = lax.axis_index("c")
    s = lax.axis_index("s")
    wid = c * NS + s
    rows = pl.ds(s * RPT, RPT)
    pltpu.sync_copy(zero_hbm.at[rows], acc.at[rows])
    pltpu.sync_copy(src_hbm.at[pl.ds(wid * kch, kch)], src_v)
    pltpu.sync_copy(dst_hbm.at[pl.ds(wid * kch, kch)], dst_v)
    plsc.subcore_barrier()

    pltpu.async_copy(g_hbm.at[src_v.at[0]], buf0, sem0)

    def body(i, carry):
        j0 = 2 * i
        j1 = 2 * i + 1
        cp1 = pltpu.async_copy(g_hbm.at[src_v.at[j1]], buf1, sem1)
        pltpu.make_async_copy(g_hbm.at[src_v.at[j0]], buf0, sem0).wait()
        pltpu.sync_copy(buf0, acc.at[dst_v.at[j0]], add=True)

        @pl.when(j1 + 1 < kch)
        def _():
            pltpu.async_copy(g_hbm.at[src_v.at[j1 + 1]], buf0, sem0)

        cp1.wait()
        pltpu.sync_copy(buf1, acc.at[dst_v.at[j1]], add=True)
        return carry

    lax.fori_loop(0, kch // 2, body, 0)
    plsc.subcore_barrier()
    pltpu.sync_copy(acc.at[rows], out_hbm.at[c].at[rows])


def _sc_aggregate(g, src_rows, dst_rows, kch, d):
    kern = functools.partial(
        pl.kernel,
        out_type=jax.ShapeDtypeStruct((NC, N_PAD, d), _f32),
        mesh=_mesh,
        scratch_types=[
            pltpu.VMEM((kch, CHUNK), jnp.int32),
            pltpu.VMEM((kch, CHUNK), jnp.int32),
            pltpu.VMEM((CHUNK, d), _f32),
            pltpu.VMEM((CHUNK, d), _f32),
            pltpu.VMEM_SHARED((N_PAD, d), _f32),
            pltpu.SemaphoreType.DMA,
            pltpu.SemaphoreType.DMA,
        ],
        compiler_params=_sc_params,
    )(functools.partial(_agg_body, kch, d))
    zeros = jnp.zeros((N_PAD, d), _f32)
    return kern(g, src_rows, dst_rows, zeros)


def _dinv_of(degp_ref):
    p = degp_ref[...]
    deg = p[0, :, 0:1] + p[1, :, 0:1] + 1.0
    return lax.rsqrt(deg)


def _layer1_body(x_ref, w1_ref, degp_ref, g1_ref):
    dinv = _dinv_of(degp_ref)
    h = jnp.dot(x_ref[...], w1_ref[...], preferred_element_type=_f32)
    g1_ref[...] = h * dinv


def _layer2_body(aggp_ref, g1_ref, degp_ref, b1_ref, w2_ref, g2_ref):
    dinv = _dinv_of(degp_ref)
    agg = aggp_ref[0] + aggp_ref[1] + g1_ref[...]
    out1 = agg * dinv + b1_ref[...]
    r = jnp.maximum(out1, 0.0)
    h2 = jnp.dot(r, w2_ref[...], preferred_element_type=_f32)
    g2_ref[...] = h2 * dinv


def _final_body(aggp_ref, g2_ref, degp_ref, b2_ref, out_ref):
    dinv = _dinv_of(degp_ref)
    agg = aggp_ref[0] + aggp_ref[1] + g2_ref[...]
    out_ref[...] = agg * dinv + b2_ref[...]


def _tc_layer1(x_pad, W1, degp):
    return pl.pallas_call(
        _layer1_body,
        out_shape=jax.ShapeDtypeStruct((N_PAD, D_H), _f32),
    )(x_pad, W1, degp)


def _tc_layer2(aggp1, g1, degp, b1, W2):
    return pl.pallas_call(
        _layer2_body,
        out_shape=jax.ShapeDtypeStruct((N_PAD, D_OUT), _f32),
    )(aggp1, g1, degp, b1, W2)


def _tc_final(aggp2, g2, degp, b2):
    return pl.pallas_call(
        _final_body,
        out_shape=jax.ShapeDtypeStruct((N_PAD, D_OUT), _f32),
    )(aggp2, g2, degp, b2)


def kernel(x, edge_index, W1, b1, W2, b2):
    n, e = x.shape[0], edge_index.shape[1]
    kch8 = -(-e // (NW * CHUNK * 8)) * 8
    epw = kch8 * CHUNK
    e_pad = NW * epw
    kch = epw // CHUNK

    pad = jnp.asarray(N_NODES + (np.arange(e_pad - e, dtype=np.int32)
                                 % (N_PAD - N_NODES)))
    src_rows = jnp.concatenate([edge_index[0], pad]).reshape(-1, CHUNK)
    dst_rows = jnp.concatenate([edge_index[1], pad]).reshape(-1, CHUNK)

    x_pad = jnp.zeros((N_PAD, D_IN), _f32).at[:n].set(x)

    degp = _sc_degree(dst_rows, kch)
    g1 = _tc_layer1(x_pad, W1, degp)
    aggp1 = _sc_aggregate(g1, src_rows, dst_rows, kch, D_H)
    g2 = _tc_layer2(aggp1, g1, degp, b1.reshape(1, D_H), W2)
    aggp2 = _sc_aggregate(g2, src_rows, dst_rows, kch, D_OUT)
    out = _tc_final(aggp2, g2, degp, b2.reshape(1, D_OUT))
    return out[:n]
